# baseline (speedup 1.0000x reference)
import functools

import jax
import jax.numpy as jnp
from jax import lax
from jax.experimental import pallas as pl
from jax.experimental.pallas import tpu as pltpu
from jax.experimental.pallas import tpu_sc as plsc

NUM_FIELDS = 26
EMBED = 16
BATCH = 16384
FIELD_SIZE = 100000
NUM_WORKERS = 32
PER_W = BATCH // NUM_WORKERS
CHUNK = 128
NCHUNK = PER_W // CHUNK
BLK = 16
NBLK = CHUNK // BLK


@functools.partial(
    pl.kernel,
    mesh=plsc.VectorSubcoreMesh(core_axis_name="c", subcore_axis_name="s"),
    out_type=jax.ShapeDtypeStruct((BATCH,), jnp.float32),
    compiler_params=pltpu.CompilerParams(
        needs_layout_passes=False, use_tc_tiling_on_sc=False),
    scratch_types=[
        pltpu.VMEM((NUM_FIELDS, CHUNK), jnp.int32),
        pltpu.VMEM((NUM_FIELDS, CHUNK), jnp.int32),
        pltpu.VMEM((NUM_FIELDS, CHUNK, EMBED), jnp.float32),
        pltpu.VMEM((NUM_FIELDS, CHUNK), jnp.float32),
        pltpu.VMEM((16,), jnp.float32),
        pltpu.VMEM((CHUNK,), jnp.float32),
        pltpu.SemaphoreType.DMA,
    ],
)
def _fm_kernel(xT_hbm, emb_hbm, fc_hbm, bias_hbm, out_hbm,
               xT_v, idx_v, rows_v, fcv_v, bias_v, out_v, sem):
    cid = lax.axis_index("c")
    sid = lax.axis_index("s")
    wid = sid * 2 + cid
    base = wid * PER_W

    pltpu.sync_copy(bias_hbm, bias_v)

    def chunk_body(c, _):
        cbase = base + c * CHUNK
        pltpu.sync_copy(xT_hbm.at[:, pl.ds(cbase, CHUNK)], xT_v)

        def idx_body(f, _):
            off = f * FIELD_SIZE

            def p_body(p, _):
                idx_v[f, pl.ds(p * 16, 16)] = xT_v[f, pl.ds(p * 16, 16)] + off
                return 0

            lax.fori_loop(0, CHUNK // 16, p_body, 0, unroll=True)
            return 0

        lax.fori_loop(0, NUM_FIELDS, idx_body, 0)

        descs = []
        for f in range(NUM_FIELDS):
            descs.append(
                pltpu.async_copy(emb_hbm.at[idx_v.at[f]], rows_v.at[f], sem))
            descs.append(
                pltpu.async_copy(fc_hbm.at[idx_v.at[f]], fcv_v.at[f], sem))
        for d in descs:
            d.wait()

        bvec = bias_v[...]

        def blk_body(blk, _):
            eb = blk * BLK
            lacc = bvec
            for f in range(NUM_FIELDS):
                lacc = lacc + fcv_v[f, pl.ds(eb, BLK)]

            lanes = lax.iota(jnp.int32, 16)

            def e_body(e, facc):
                v = rows_v[0, eb + e, :]
                s = v
                ss = v * v
                for f in range(1, NUM_FIELDS):
                    v = rows_v[f, eb + e, :]
                    s = s + v
                    ss = ss + v * v
                t = s * s - ss
                return jnp.where(lanes == e, 0.5 * jnp.sum(t), facc)

            facc = lax.fori_loop(0, BLK, e_body,
                                 jnp.zeros((16,), jnp.float32))

            z = lacc + facc
            out_v[pl.ds(eb, BLK)] = 1.0 / (1.0 + jnp.exp(-z))
            return 0

        lax.fori_loop(0, NBLK, blk_body, 0)

        pltpu.sync_copy(out_v, out_hbm.at[pl.ds(cbase, CHUNK)])
        return 0

    lax.fori_loop(0, NCHUNK, chunk_body, 0)


def kernel(x, emb_table, fc_table, bias):
    xT = x.astype(jnp.int32).T
    fc = fc_table.reshape(-1)
    bias16 = jnp.broadcast_to(bias.astype(jnp.float32), (16,))
    return _fm_kernel(xT, emb_table, fc, bias16)
